# baseline (speedup 1.0000x reference)
import jax
import jax.numpy as jnp
from jax.experimental import pallas as pl
from jax.experimental.pallas import tpu as pltpu

_EPS = 1e-3
_CHUNK = 512


def _bn_kernel(nvv_ref, x_hbm, gamma_ref, beta_ref, mean_ref, var_ref, out_hbm,
               xs_ref, ys_ref, sem_in, sem_out):
    B, n, fd = x_hbm.shape
    nchunk = n // _CHUNK

    def issue_in(b, slot):
        nv = nvv_ref[b]
        for c in range(nchunk):
            @pl.when(c * _CHUNK < nv)
            def _copy(b=b, c=c, slot=slot):
                pltpu.make_async_copy(
                    x_hbm.at[b, pl.ds(c * _CHUNK, _CHUNK), :],
                    xs_ref.at[slot, pl.ds(c * _CHUNK, _CHUNK), :],
                    sem_in.at[slot, c],
                ).start()

    def wait_in(b, slot):
        nv = nvv_ref[b]
        for c in range(nchunk):
            @pl.when(c * _CHUNK < nv)
            def _wait(c=c, slot=slot):
                pltpu.make_async_copy(
                    x_hbm.at[0, pl.ds(0, _CHUNK), :],
                    xs_ref.at[slot, pl.ds(0, _CHUNK), :],
                    sem_in.at[slot, c],
                ).wait()

    def issue_out(b, slot):
        pltpu.make_async_copy(
            ys_ref.at[slot], out_hbm.at[b], sem_out.at[slot]).start()

    def wait_out(b, slot):
        pltpu.make_async_copy(
            ys_ref.at[slot], out_hbm.at[b], sem_out.at[slot]).wait()

    scale = gamma_ref[0] * jax.lax.rsqrt(var_ref[0] + _EPS)
    bias = beta_ref[0] - mean_ref[0] * scale
    row = jax.lax.broadcasted_iota(jnp.int32, (n, 1), 0)

    issue_in(0, 0)
    for b in range(B):
        slot = b % 2
        if b + 1 < B:
            issue_in(b + 1, (b + 1) % 2)
        wait_in(b, slot)
        if b >= 2:
            wait_out(b - 2, slot)
        nv = nvv_ref[b]
        y = xs_ref[slot] * scale[None, :] + bias[None, :]
        ys_ref[slot] = jnp.where(row < nv, y, jnp.zeros_like(y))
        issue_out(b, slot)
    wait_out(B - 2, (B - 2) % 2)
    wait_out(B - 1, (B - 1) % 2)


def kernel(voxel_features, num_valid_voxels, gamma, beta, moving_mean, moving_var):
    B, N, FD = voxel_features.shape

    def param_map(i, nvv):
        return (0, 0)

    grid_spec = pltpu.PrefetchScalarGridSpec(
        num_scalar_prefetch=1,
        grid=(1,),
        in_specs=[
            pl.BlockSpec(memory_space=pl.ANY),
            pl.BlockSpec((1, FD), param_map),
            pl.BlockSpec((1, FD), param_map),
            pl.BlockSpec((1, FD), param_map),
            pl.BlockSpec((1, FD), param_map),
        ],
        out_specs=pl.BlockSpec(memory_space=pl.ANY),
        scratch_shapes=[
            pltpu.VMEM((2, N, FD), jnp.float32),
            pltpu.VMEM((2, N, FD), jnp.float32),
            pltpu.SemaphoreType.DMA((2, N // _CHUNK)),
            pltpu.SemaphoreType.DMA((2,)),
        ],
    )

    return pl.pallas_call(
        _bn_kernel,
        grid_spec=grid_spec,
        out_shape=jax.ShapeDtypeStruct((B, N, FD), voxel_features.dtype),
        compiler_params=pltpu.CompilerParams(
            dimension_semantics=("arbitrary",),
        ),
    )(
        num_valid_voxels,
        voxel_features,
        gamma.reshape(1, FD),
        beta.reshape(1, FD),
        moving_mean.reshape(1, FD),
        moving_var.reshape(1, FD),
    )


# speedup vs baseline: 1.2218x; 1.2218x over previous
import jax
import jax.numpy as jnp
from jax.experimental import pallas as pl
from jax.experimental.pallas import tpu as pltpu

_EPS = 1e-3
_CHUNK = 512
_DEPTH = 4


def _bn_kernel(nvv_ref, x_hbm, gamma_ref, beta_ref, mean_ref, var_ref, out_hbm,
               xs_ref, ys_ref, sem_in, sem_out):
    B, n, fd = x_hbm.shape
    nchunk = n // _CHUNK

    def issue_in(b, slot):
        nv = nvv_ref[b]
        for c in range(nchunk):
            @pl.when(c * _CHUNK < nv)
            def _copy(b=b, c=c, slot=slot):
                pltpu.make_async_copy(
                    x_hbm.at[b, pl.ds(c * _CHUNK, _CHUNK), :],
                    xs_ref.at[slot, pl.ds(c * _CHUNK, _CHUNK), :],
                    sem_in.at[slot, c],
                ).start()

    def wait_in(b, slot):
        nv = nvv_ref[b]
        for c in range(nchunk):
            @pl.when(c * _CHUNK < nv)
            def _wait(c=c, slot=slot):
                pltpu.make_async_copy(
                    x_hbm.at[0, pl.ds(0, _CHUNK), :],
                    xs_ref.at[slot, pl.ds(0, _CHUNK), :],
                    sem_in.at[slot, c],
                ).wait()

    def issue_out(b, slot):
        pltpu.make_async_copy(
            ys_ref.at[slot], out_hbm.at[b], sem_out.at[slot]).start()

    def wait_out(b, slot):
        pltpu.make_async_copy(
            ys_ref.at[slot], out_hbm.at[b], sem_out.at[slot]).wait()

    scale = gamma_ref[0] * jax.lax.rsqrt(var_ref[0] + _EPS)
    bias = beta_ref[0] - mean_ref[0] * scale
    row = jax.lax.broadcasted_iota(jnp.int32, (n, 1), 0)

    for p in range(_DEPTH - 1):
        issue_in(p, p)
    for b in range(B):
        slot = b % _DEPTH
        if b + _DEPTH - 1 < B:
            issue_in(b + _DEPTH - 1, (b + _DEPTH - 1) % _DEPTH)
        wait_in(b, slot)
        if b >= _DEPTH:
            wait_out(b - _DEPTH, slot)
        nv = nvv_ref[b]
        y = xs_ref[slot] * scale[None, :] + bias[None, :]
        ys_ref[slot] = jnp.where(row < nv, y, jnp.zeros_like(y))
        issue_out(b, slot)
    for b in range(B - _DEPTH, B):
        wait_out(b, b % _DEPTH)


def kernel(voxel_features, num_valid_voxels, gamma, beta, moving_mean, moving_var):
    B, N, FD = voxel_features.shape

    def param_map(i, nvv):
        return (0, 0)

    grid_spec = pltpu.PrefetchScalarGridSpec(
        num_scalar_prefetch=1,
        grid=(1,),
        in_specs=[
            pl.BlockSpec(memory_space=pl.ANY),
            pl.BlockSpec((1, FD), param_map),
            pl.BlockSpec((1, FD), param_map),
            pl.BlockSpec((1, FD), param_map),
            pl.BlockSpec((1, FD), param_map),
        ],
        out_specs=pl.BlockSpec(memory_space=pl.ANY),
        scratch_shapes=[
            pltpu.VMEM((_DEPTH, N, FD), jnp.float32),
            pltpu.VMEM((_DEPTH, N, FD), jnp.float32),
            pltpu.SemaphoreType.DMA((_DEPTH, N // _CHUNK)),
            pltpu.SemaphoreType.DMA((_DEPTH,)),
        ],
    )

    return pl.pallas_call(
        _bn_kernel,
        grid_spec=grid_spec,
        out_shape=jax.ShapeDtypeStruct((B, N, FD), voxel_features.dtype),
        compiler_params=pltpu.CompilerParams(
            dimension_semantics=("arbitrary",),
        ),
    )(
        num_valid_voxels,
        voxel_features,
        gamma.reshape(1, FD),
        beta.reshape(1, FD),
        moving_mean.reshape(1, FD),
        moving_var.reshape(1, FD),
    )


# speedup vs baseline: 1.2828x; 1.0499x over previous
import jax
import jax.numpy as jnp
from jax.experimental import pallas as pl
from jax.experimental.pallas import tpu as pltpu

_EPS = 1e-3
_CHUNK = 512
_DEPTH = 8


def _bn_kernel(nvv_ref, x_hbm, gamma_ref, beta_ref, mean_ref, var_ref, out_hbm,
               xs_ref, ys_ref, sem_in, sem_out):
    B, n, fd = x_hbm.shape
    nchunk = n // _CHUNK

    def issue_in(b, slot):
        nv = nvv_ref[b]
        for c in range(nchunk):
            @pl.when(c * _CHUNK < nv)
            def _copy(b=b, c=c, slot=slot):
                pltpu.make_async_copy(
                    x_hbm.at[b, pl.ds(c * _CHUNK, _CHUNK), :],
                    xs_ref.at[slot, pl.ds(c * _CHUNK, _CHUNK), :],
                    sem_in.at[slot, c],
                ).start()

    def wait_in(b, slot):
        nv = nvv_ref[b]
        for c in range(nchunk):
            @pl.when(c * _CHUNK < nv)
            def _wait(c=c, slot=slot):
                pltpu.make_async_copy(
                    x_hbm.at[0, pl.ds(0, _CHUNK), :],
                    xs_ref.at[slot, pl.ds(0, _CHUNK), :],
                    sem_in.at[slot, c],
                ).wait()

    def issue_out(b, slot):
        pltpu.make_async_copy(
            ys_ref.at[slot], out_hbm.at[b], sem_out.at[slot]).start()

    def wait_out(b, slot):
        pltpu.make_async_copy(
            ys_ref.at[slot], out_hbm.at[b], sem_out.at[slot]).wait()

    scale = gamma_ref[0] * jax.lax.rsqrt(var_ref[0] + _EPS)
    bias = beta_ref[0] - mean_ref[0] * scale
    row = jax.lax.broadcasted_iota(jnp.int32, (n, 1), 0)

    for p in range(_DEPTH - 1):
        issue_in(p, p)
    for b in range(B):
        slot = b % _DEPTH
        if b + _DEPTH - 1 < B:
            issue_in(b + _DEPTH - 1, (b + _DEPTH - 1) % _DEPTH)
        wait_in(b, slot)
        if b >= _DEPTH:
            wait_out(b - _DEPTH, slot)
        nv = nvv_ref[b]
        y = xs_ref[slot] * scale[None, :] + bias[None, :]
        ys_ref[slot] = jnp.where(row < nv, y, jnp.zeros_like(y))
        issue_out(b, slot)
    for b in range(B - _DEPTH, B):
        wait_out(b, b % _DEPTH)


def kernel(voxel_features, num_valid_voxels, gamma, beta, moving_mean, moving_var):
    B, N, FD = voxel_features.shape

    def param_map(i, nvv):
        return (0, 0)

    grid_spec = pltpu.PrefetchScalarGridSpec(
        num_scalar_prefetch=1,
        grid=(1,),
        in_specs=[
            pl.BlockSpec(memory_space=pl.ANY),
            pl.BlockSpec((1, FD), param_map),
            pl.BlockSpec((1, FD), param_map),
            pl.BlockSpec((1, FD), param_map),
            pl.BlockSpec((1, FD), param_map),
        ],
        out_specs=pl.BlockSpec(memory_space=pl.ANY),
        scratch_shapes=[
            pltpu.VMEM((_DEPTH, N, FD), jnp.float32),
            pltpu.VMEM((_DEPTH, N, FD), jnp.float32),
            pltpu.SemaphoreType.DMA((_DEPTH, N // _CHUNK)),
            pltpu.SemaphoreType.DMA((_DEPTH,)),
        ],
    )

    return pl.pallas_call(
        _bn_kernel,
        grid_spec=grid_spec,
        out_shape=jax.ShapeDtypeStruct((B, N, FD), voxel_features.dtype),
        compiler_params=pltpu.CompilerParams(
            dimension_semantics=("arbitrary",),
        ),
    )(
        num_valid_voxels,
        voxel_features,
        gamma.reshape(1, FD),
        beta.reshape(1, FD),
        moving_mean.reshape(1, FD),
        moving_var.reshape(1, FD),
    )
